# baseline (speedup 1.0000x reference)
import functools

import jax
import jax.numpy as jnp
import numpy as np
from jax import lax
from jax.experimental import pallas as pl
from jax.experimental.pallas import tpu as pltpu
from jax.experimental.pallas import tpu_sc as plsc

DIM = 100
PADW = 112
GRP = 128
CW = GRP * DIM
PERIOD = 400
RPP = PERIOD // DIM
VPP = PERIOD // 16
NGRP = CW // PERIOD


def _patterns():
    w = np.arange(PERIOD, dtype=np.int32)
    return np.concatenate([w // DIM, w % DIM])


@functools.cache
def _make_kernel(n_total):
    info = plsc.get_sparse_core_info()
    nc, ns = info.num_cores, info.num_subcores
    nw = nc * ns
    per_w = n_total // nw
    n_chunks = per_w // GRP
    assert per_w % GRP == 0

    def body(x_hbm, table_hbm, patt_hbm, out_hbm, idx_v, rows_p, rows_c, patt_v, sem):
        wid = lax.axis_index("s") * nc + lax.axis_index("c")
        base = wid * n_chunks

        pltpu.sync_copy(patt_hbm, patt_v)

        def compact(g, carry):
            gr = g * RPP
            for k in range(VPP):
                rv = patt_v[pl.ds(16 * k, 16)] + gr
                cv = patt_v[pl.ds(PERIOD + 16 * k, 16)]
                rows_c[pl.ds(g * PERIOD + 16 * k, 16)] = plsc.load_gather(
                    rows_p, [rv, cv]
                )
            return carry

        def step(g, carry):
            xrow = base + g
            pltpu.sync_copy(x_hbm.at[xrow], idx_v)
            pltpu.async_copy(table_hbm.at[idx_v], rows_p, sem).wait()
            lax.fori_loop(0, NGRP, compact, 0)
            pltpu.sync_copy(rows_c, out_hbm.at[pl.ds(xrow * CW, CW)])
            return carry

        lax.fori_loop(0, n_chunks, step, 0)

    mesh = plsc.VectorSubcoreMesh(core_axis_name="c", subcore_axis_name="s")
    return pl.kernel(
        body,
        out_type=jax.ShapeDtypeStruct((n_total * DIM,), jnp.float32),
        mesh=mesh,
        compiler_params=pltpu.CompilerParams(
            use_tc_tiling_on_sc=False, needs_layout_passes=False
        ),
        scratch_types=[
            pltpu.VMEM((GRP,), jnp.int32),
            pltpu.VMEM((GRP, PADW), jnp.float32),
            pltpu.VMEM((CW,), jnp.float32),
            pltpu.VMEM((2 * PERIOD,), jnp.int32),
            pltpu.SemaphoreType.DMA,
        ],
    )


def kernel(x, table):
    b, s = x.shape
    n = b * s
    xf = x.reshape(n // GRP, GRP).astype(jnp.int32)
    tp = jnp.pad(table, ((0, 0), (0, PADW - DIM)))
    patt = jnp.asarray(_patterns())
    out = _make_kernel(n)(xf, tp, patt)
    return out.reshape(b, s, DIM)


# speedup vs baseline: 1.1476x; 1.1476x over previous
import functools

import jax
import jax.numpy as jnp
import numpy as np
from jax import lax
from jax.experimental import pallas as pl
from jax.experimental.pallas import tpu as pltpu
from jax.experimental.pallas import tpu_sc as plsc

DIM = 100
PADW = 112
GRP = 128
CW = GRP * DIM
PERIOD = 400
RPP = PERIOD // DIM
VPP = PERIOD // 16
NGRP = CW // PERIOD


def _patterns():
    w = np.arange(PERIOD, dtype=np.int32)
    return np.concatenate([w // DIM, w % DIM])


@functools.cache
def _make_kernel(n_total):
    info = plsc.get_sparse_core_info()
    nc, ns = info.num_cores, info.num_subcores
    nw = nc * ns
    per_w = n_total // nw
    n_chunks = per_w // GRP
    assert per_w % GRP == 0 and n_chunks >= 4

    def body(x_hbm, table_hbm, patt_hbm, out_hbm, idx_all, p0, p1, c0, c1,
             patt_v, sg0, sg1, so0, so1):
        rows_p = (p0, p1)
        rows_c = (c0, c1)
        sem_g = (sg0, sg1)
        sem_o = (so0, so1)
        wid = lax.axis_index("s") * nc + lax.axis_index("c")
        base = wid * n_chunks

        pltpu.sync_copy(patt_hbm, patt_v)
        pltpu.sync_copy(x_hbm.at[pl.ds(base, n_chunks)], idx_all)

        def compact(b):
            def grp(g, carry):
                gr = g * RPP
                for k in range(VPP):
                    rv = patt_v[pl.ds(16 * k, 16)] + gr
                    cv = patt_v[pl.ds(PERIOD + 16 * k, 16)]
                    rows_c[b][pl.ds(g * PERIOD + 16 * k, 16)] = plsc.load_gather(
                        rows_p[b], [rv, cv]
                    )
                return carry

            lax.fori_loop(0, NGRP, grp, 0)

        def start_gather(c, b):
            return pltpu.async_copy(table_hbm.at[idx_all.at[c]], rows_p[b], sem_g[b])

        def wait_gather(b):
            pltpu.make_async_copy(table_hbm.at[idx_all.at[0]], rows_p[b],
                                  sem_g[b]).wait()

        def start_out(c, b):
            return pltpu.async_copy(
                rows_c[b], out_hbm.at[pl.ds((base + c) * CW, CW)], sem_o[b]
            )

        def wait_out(b):
            pltpu.make_async_copy(rows_c[b],
                                  out_hbm.at[pl.ds(base * CW, CW)], sem_o[b]).wait()

        def handle(c, b, start_next, do_wait_out):
            wait_gather(b)
            if start_next:
                start_gather(c + 1, 1 - b)
            if do_wait_out:
                wait_out(b)
            compact(b)
            start_out(c, b)

        start_gather(0, 0)
        handle(0, 0, True, False)
        handle(1, 1, True, False)

        def outer(g2, carry):
            handle(2 * g2, 0, True, True)
            handle(2 * g2 + 1, 1, True, True)
            return carry

        lax.fori_loop(1, n_chunks // 2 - 1, outer, 0)

        handle(n_chunks - 2, 0, True, True)
        handle(n_chunks - 1, 1, False, True)
        wait_out(0)
        wait_out(1)

    mesh = plsc.VectorSubcoreMesh(core_axis_name="c", subcore_axis_name="s")
    return pl.kernel(
        body,
        out_type=jax.ShapeDtypeStruct((n_total * DIM,), jnp.float32),
        mesh=mesh,
        compiler_params=pltpu.CompilerParams(
            use_tc_tiling_on_sc=False, needs_layout_passes=False
        ),
        scratch_types=[
            pltpu.VMEM((n_total // nw // GRP, GRP), jnp.int32),
            pltpu.VMEM((GRP, PADW), jnp.float32),
            pltpu.VMEM((GRP, PADW), jnp.float32),
            pltpu.VMEM((CW,), jnp.float32),
            pltpu.VMEM((CW,), jnp.float32),
            pltpu.VMEM((2 * PERIOD,), jnp.int32),
            pltpu.SemaphoreType.DMA,
            pltpu.SemaphoreType.DMA,
            pltpu.SemaphoreType.DMA,
            pltpu.SemaphoreType.DMA,
        ],
    )


def kernel(x, table):
    b, s = x.shape
    n = b * s
    xf = x.reshape(n // GRP, GRP).astype(jnp.int32)
    tp = jnp.pad(table, ((0, 0), (0, PADW - DIM)))
    patt = jnp.asarray(_patterns())
    out = _make_kernel(n)(xf, tp, patt)
    return out.reshape(b, s, DIM)


# speedup vs baseline: 2.2245x; 1.9384x over previous
import functools

import jax
import jax.numpy as jnp
import numpy as np
from jax import lax
from jax.experimental import pallas as pl
from jax.experimental.pallas import tpu as pltpu
from jax.experimental.pallas import tpu_sc as plsc

DIM = 100
PADW = 112
GRP = 128
RPP = 4
NGRP = GRP // RPP


def _patterns():
    tr = np.repeat(np.arange(4, dtype=np.int32), 4)
    tc = (96 + np.tile(np.arange(4, dtype=np.int32), 4)).astype(np.int32)
    return np.concatenate([tr, tc])


@functools.cache
def _make_kernel(n_total):
    info = plsc.get_sparse_core_info()
    nc, ns = info.num_cores, info.num_subcores
    nw = nc * ns
    per_w = n_total // nw
    n_chunks = per_w // GRP
    assert per_w % GRP == 0 and n_chunks >= 4

    def body(x_hbm, table_hbm, patt_hbm, out_hbm, idx_all, p0, p1, c0, c1,
             patt_v, sg0, sg1, so0, so1):
        rows_p = (p0, p1)
        rows_c = (c0, c1)
        sem_g = (sg0, sg1)
        sem_o = (so0, so1)
        wid = lax.axis_index("s") * nc + lax.axis_index("c")
        base = wid * n_chunks

        pltpu.sync_copy(patt_hbm, patt_v)
        pltpu.sync_copy(x_hbm.at[pl.ds(base, n_chunks)], idx_all)

        tr = patt_v[pl.ds(0, 16)]
        tc = patt_v[pl.ds(16, 16)]

        def compact(b):
            def grp(g, carry):
                r0 = g * RPP
                for d in range(RPP):
                    r = r0 + d
                    for c0 in range(0, 96, 16):
                        rows_c[b][r, pl.ds(c0, 16)] = rows_p[b][r, pl.ds(c0, 16)]
                rv = tr + r0
                v = plsc.load_gather(rows_p[b], [rv, tc])
                plsc.store_scatter(rows_c[b], [rv, tc], v)
                return carry

            lax.fori_loop(0, NGRP, grp, 0)

        def start_gather(c, b):
            return pltpu.async_copy(table_hbm.at[idx_all.at[c]], rows_p[b], sem_g[b])

        def wait_gather(b):
            pltpu.make_async_copy(table_hbm.at[idx_all.at[0]], rows_p[b],
                                  sem_g[b]).wait()

        def start_out(c, b):
            return pltpu.async_copy(
                rows_c[b], out_hbm.at[pl.ds((base + c) * GRP, GRP)], sem_o[b]
            )

        def wait_out(b):
            pltpu.make_async_copy(rows_c[b],
                                  out_hbm.at[pl.ds(base * GRP, GRP)], sem_o[b]).wait()

        def handle(c, b, start_next, do_wait_out):
            wait_gather(b)
            if start_next:
                start_gather(c + 1, 1 - b)
            if do_wait_out:
                wait_out(b)
            compact(b)
            start_out(c, b)

        start_gather(0, 0)
        handle(0, 0, True, False)
        handle(1, 1, True, False)

        def outer(g2, carry):
            handle(2 * g2, 0, True, True)
            handle(2 * g2 + 1, 1, True, True)
            return carry

        lax.fori_loop(1, n_chunks // 2 - 1, outer, 0)

        handle(n_chunks - 2, 0, True, True)
        handle(n_chunks - 1, 1, False, True)
        wait_out(0)
        wait_out(1)

    mesh = plsc.VectorSubcoreMesh(core_axis_name="c", subcore_axis_name="s")
    return pl.kernel(
        body,
        out_type=jax.ShapeDtypeStruct((n_total, DIM), jnp.float32),
        mesh=mesh,
        compiler_params=pltpu.CompilerParams(
            use_tc_tiling_on_sc=False, needs_layout_passes=False
        ),
        scratch_types=[
            pltpu.VMEM((n_total // nw // GRP, GRP), jnp.int32),
            pltpu.VMEM((GRP, PADW), jnp.float32),
            pltpu.VMEM((GRP, PADW), jnp.float32),
            pltpu.VMEM((GRP, DIM), jnp.float32),
            pltpu.VMEM((GRP, DIM), jnp.float32),
            pltpu.VMEM((32,), jnp.int32),
            pltpu.SemaphoreType.DMA,
            pltpu.SemaphoreType.DMA,
            pltpu.SemaphoreType.DMA,
            pltpu.SemaphoreType.DMA,
        ],
    )


def kernel(x, table):
    b, s = x.shape
    n = b * s
    xf = x.reshape(n // GRP, GRP).astype(jnp.int32)
    tp = jnp.pad(table, ((0, 0), (0, PADW - DIM)))
    patt = jnp.asarray(_patterns())
    out = _make_kernel(n)(xf, tp, patt)
    return out.reshape(b, s, DIM)
